# baseline (speedup 1.0000x reference)
import functools

import jax
import jax.numpy as jnp
from jax import lax
from jax.experimental import pallas as pl
from jax.experimental.pallas import tpu as pltpu
from jax.experimental.pallas import tpu_sc as plsc

N = 50000
E = 800000
HALF = 25000
OSTR = 25088
NPAD = 2 * OSTR
ACC_ROWS = 25216
TRASH = 25088
NBLK = 196
NBLK_ACC = 197
EPT = 819200
NBUF = 3

f32 = jnp.float32


def _agg_body(W, NSTR, SPT, h, srcr, dstr, out, src_i, dst_i, rows, acc, gs, ss):
    c = lax.axis_index("c")
    s = lax.axis_index("s")
    cbase = c * HALF
    trash = TRASH + s
    zero = jnp.zeros((16,), f32)

    def zrow(i, carry):
        for k in range(W // 16):
            rows[0][i, pl.ds(k * 16, 16)] = zero
        return carry

    lax.fori_loop(0, 128, zrow, 0)

    def zblk(m, carry):
        blk = m * 16 + s

        @pl.when(blk < NBLK_ACC)
        def _():
            pltpu.sync_copy(rows[0], acc.at[pl.ds(blk * 128, 128)])

        return carry

    lax.fori_loop(0, 13, zblk, 0)
    plsc.subcore_barrier()

    def super_body(j, carry):
        g = s * SPT + j
        pltpu.sync_copy(srcr.at[g], src_i)
        pltpu.sync_copy(dstr.at[g], dst_i)
        for i in range(NSTR):
            for k in range(8):
                sl = (i, pl.ds(k * 16, 16))
                sv = src_i[sl]
                src_i[sl] = sv + jnp.where(sv >= HALF, 88, 0)
                d = dst_i[sl] - cbase
                ok = (d >= 0) & (d < HALF)
                dst_i[sl] = jnp.where(ok, d, trash)
        if W == 64:
            for i in range(NSTR):
                pltpu.async_copy(h.at[src_i.at[i]], rows[0], gs[0]).wait()
                pltpu.sync_copy(rows[0], acc.at[dst_i.at[i]], add=True)
        else:
            nbuf = len(rows)
            gh = [None] * nbuf
            sh = [None] * nbuf
            gh[0] = pltpu.async_copy(h.at[src_i.at[0]], rows[0], gs[0])
            for i in range(NSTR):
                b = i % nbuf
                if i + 1 < NSTR:
                    bn = (i + 1) % nbuf
                    if sh[bn] is not None:
                        sh[bn].wait()
                    gh[bn] = pltpu.async_copy(h.at[src_i.at[i + 1]], rows[bn], gs[bn])
                gh[b].wait()
                sh[b] = pltpu.async_copy(rows[b], acc.at[dst_i.at[i]], ss[b], add=True)
            for b in range(nbuf):
                if sh[b] is not None:
                    sh[b].wait()
        return carry

    lax.fori_loop(0, SPT, super_body, 0)
    plsc.subcore_barrier()

    def oblk(m, carry):
        blk = m * 16 + s

        @pl.when(blk < NBLK)
        def _():
            pltpu.sync_copy(acc.at[pl.ds(blk * 128, 128)],
                            out.at[pl.ds(c * OSTR + blk * 128, 128)])

        return carry

    lax.fori_loop(0, 13, oblk, 0)


def _make_agg(W):
    mesh = plsc.VectorSubcoreMesh(core_axis_name="c", subcore_axis_name="s")
    NSTR = 16 if W == 16 else 8
    SPT = EPT // (NSTR * 128) // 16

    @functools.partial(
        pl.kernel,
        out_type=jax.ShapeDtypeStruct((NPAD, W), f32),
        mesh=mesh,
        compiler_params=pltpu.CompilerParams(use_tc_tiling_on_sc=False),
        scratch_types=[
            pltpu.VMEM((NSTR, 128), jnp.int32),
            pltpu.VMEM((NSTR, 128), jnp.int32),
        ] + [pltpu.VMEM((128, W), f32) for _ in range(NBUF)] + [
            pltpu.VMEM_SHARED((ACC_ROWS, W), f32),
        ] + [pltpu.SemaphoreType.DMA for _ in range(2 * NBUF)],
    )
    def agg(h, srcr, dstr, out, src_i, dst_i, r0, r1, r2, acc,
            g0, g1, g2, s0, s1, s2):
        _agg_body(W, NSTR, SPT, h, srcr, dstr, out, src_i, dst_i, [r0, r1, r2],
                  acc, [g0, g1, g2], [s0, s1, s2])

    return agg


_agg16 = _make_agg(16)
_agg64 = _make_agg(64)

BLK = 3584
GRID = NPAD // BLK


def _row_spec(w):
    return pl.BlockSpec((BLK, w), lambda i: (i, 0))


def _full_spec(shape):
    return pl.BlockSpec(shape, lambda i: (0,) * len(shape))


def _l0_body(acc_ref, xp_ref, wl_ref, bl_ref, wr_ref, h_ref, inv_ref):
    acc = acc_ref[...]
    inv = 1.0 / jnp.maximum(acc[:, 13:14], 1.0)
    mean = acc * inv
    o = jnp.dot(mean, wl_ref[...], preferred_element_type=f32)
    o = o + jnp.dot(xp_ref[...], wr_ref[...], preferred_element_type=f32)
    o = o + bl_ref[...][0:1, :]
    h_ref[...] = jnp.maximum(o, 0.0)
    inv_ref[...] = inv


_l0_call = pl.pallas_call(
    _l0_body,
    grid=(GRID,),
    in_specs=[_row_spec(16), _row_spec(16), _full_spec((16, 64)),
              _full_spec((8, 64)), _full_spec((16, 64))],
    out_specs=[_row_spec(64), _row_spec(1)],
    out_shape=[jax.ShapeDtypeStruct((NPAD, 64), f32),
               jax.ShapeDtypeStruct((NPAD, 1), f32)],
)


def _mid_body(acc_ref, h_ref, inv_ref, wl_ref, bl_ref, wr_ref, o_ref):
    mean = acc_ref[...] * inv_ref[...]
    o = jnp.dot(mean, wl_ref[...], preferred_element_type=f32)
    o = o + jnp.dot(h_ref[...], wr_ref[...], preferred_element_type=f32)
    o = o + bl_ref[...][0:1, :]
    o_ref[...] = jnp.maximum(o, 0.0)


_mid_call = pl.pallas_call(
    _mid_body,
    grid=(GRID,),
    in_specs=[_row_spec(64), _row_spec(64), _row_spec(1), _full_spec((64, 64)),
              _full_spec((8, 64)), _full_spec((64, 64))],
    out_specs=_row_spec(64),
    out_shape=jax.ShapeDtypeStruct((NPAD, 64), f32),
)


def _l3_body(acc_ref, h_ref, inv_ref, wl_ref, bl_ref, wr_ref, wc_ref, bc_ref,
             o_ref, pw_ref):
    mean = acc_ref[...] * inv_ref[...]
    o = jnp.dot(mean, wl_ref[...], preferred_element_type=f32)
    o = o + jnp.dot(h_ref[...], wr_ref[...], preferred_element_type=f32)
    o = o + bl_ref[...][0:1, :]
    o = jnp.maximum(o, 0.0)
    o_ref[...] = o
    pw_ref[...] = jnp.dot(o, wc_ref[...], preferred_element_type=f32) + bc_ref[...][0:1, :]


_l3_call = pl.pallas_call(
    _l3_body,
    grid=(GRID,),
    in_specs=[_row_spec(64), _row_spec(64), _row_spec(1), _full_spec((64, 64)),
              _full_spec((8, 64)), _full_spec((64, 64)), _full_spec((64, 16)),
              _full_spec((8, 16))],
    out_specs=[_row_spec(64), _row_spec(16)],
    out_shape=[jax.ShapeDtypeStruct((NPAD, 64), f32),
               jax.ShapeDtypeStruct((NPAD, 16), f32)],
)


def _fin_body(acc_ref, pw_ref, inv_ref, o_ref):
    o_ref[...] = jax.nn.sigmoid(
        acc_ref[...][:, 0:1] * inv_ref[...] + pw_ref[...][:, 1:2])


_fin_call = pl.pallas_call(
    _fin_body,
    grid=(GRID,),
    in_specs=[_row_spec(16), _row_spec(16), _row_spec(1)],
    out_specs=_row_spec(1),
    out_shape=jax.ShapeDtypeStruct((NPAD, 1), f32),
)


def _pad_rows(a):
    w = a.shape[1]
    return (jnp.zeros((2, OSTR, w), f32)
            .at[:, :HALF].set(a.reshape(2, HALF, w))
            .reshape(NPAD, w))


def kernel(x, edge_index, Wl0, bl0, Wr0, Wl1, bl1, Wr1, Wl2, bl2, Wr2,
           Wl3, bl3, Wr3, Wl4, bl4, Wr4):
    src = edge_index[0]
    dst = edge_index[1]
    pad = EPT - E
    srcf = jnp.concatenate([src, jnp.zeros((pad,), jnp.int32)])
    dstf = jnp.concatenate([dst, jnp.full((pad,), -1, jnp.int32)])
    srcr16 = srcf.reshape(-1, 16, 128)
    dstr16 = dstf.reshape(-1, 16, 128)
    srcr64 = srcf.reshape(-1, 8, 128)
    dstr64 = dstf.reshape(-1, 8, 128)
    xp = jnp.concatenate([x, jnp.ones((N, 1), f32), jnp.zeros((N, 2), f32)], axis=1)
    xpp = _pad_rows(xp)
    wl0p = jnp.concatenate([Wl0, jnp.zeros((3, 64), f32)], axis=0)
    wr0p = jnp.concatenate([Wr0, jnp.zeros((3, 64), f32)], axis=0)
    wcat = jnp.concatenate([Wl4, Wr4, jnp.zeros((64, 14), f32)], axis=1)
    bcat = jnp.zeros((16,), f32).at[1].set(bl4[0])
    b8 = lambda b: jnp.tile(b.reshape(1, -1), (8, 1))

    acc0 = _agg16(xpp, srcr16, dstr16)
    h0, inv = _l0_call(acc0, xpp, wl0p, b8(bl0), wr0p)
    acc1 = _agg64(h0, srcr64, dstr64)
    h1 = _mid_call(acc1, h0, inv, Wl1, b8(bl1), Wr1)
    acc2 = _agg64(h1, srcr64, dstr64)
    h2 = _mid_call(acc2, h1, inv, Wl2, b8(bl2), Wr2)
    acc3 = _agg64(h2, srcr64, dstr64)
    h3, pw = _l3_call(acc3, h2, inv, Wl3, b8(bl3), Wr3, wcat, b8(bcat))
    acc4 = _agg16(pw, srcr16, dstr16)
    outp = _fin_call(acc4, pw, inv)
    return outp.reshape(2, OSTR, 1)[:, :HALF].reshape(N, 1)


# speedup vs baseline: 1.4760x; 1.4760x over previous
import functools

import jax
import jax.numpy as jnp
from jax import lax
from jax.experimental import pallas as pl
from jax.experimental.pallas import tpu as pltpu
from jax.experimental.pallas import tpu_sc as plsc

N = 50000
E = 800000
HALF = 25000
OSTR = 25088
NPAD = 2 * OSTR
ACC_ROWS = 25216
TRASH = 25088
NBLK = 196
NBLK_ACC = 197
EPT = 819200
EPT64 = 802816
CPT64 = 49
TRASH64 = 25100
NBUF = 3

f32 = jnp.float32


def _agg_body16(NSTR, SPT, h, srcr, dstr, out, src_i, dst_i, rows, acc, gs, ss):
    W = 16
    c = lax.axis_index("c")
    s = lax.axis_index("s")
    cbase = c * HALF
    trash = TRASH + s
    zero = jnp.zeros((16,), f32)

    def zrow(i, carry):
        for k in range(W // 16):
            rows[0][i, pl.ds(k * 16, 16)] = zero
        return carry

    lax.fori_loop(0, 128, zrow, 0)

    def zblk(m, carry):
        blk = m * 16 + s

        @pl.when(blk < NBLK_ACC)
        def _():
            pltpu.sync_copy(rows[0], acc.at[pl.ds(blk * 128, 128)])

        return carry

    lax.fori_loop(0, 13, zblk, 0)
    plsc.subcore_barrier()

    def super_body(j, carry):
        g = s * SPT + j
        pltpu.sync_copy(srcr.at[g], src_i)
        pltpu.sync_copy(dstr.at[g], dst_i)
        for i in range(NSTR):
            for k in range(8):
                sl = (i, pl.ds(k * 16, 16))
                sv = src_i[sl]
                src_i[sl] = sv + jnp.where(sv >= HALF, 88, 0)
                d = dst_i[sl] - cbase
                ok = (d >= 0) & (d < HALF)
                dst_i[sl] = jnp.where(ok, d, trash)
        nbuf = len(rows)
        gh = [None] * nbuf
        sh = [None] * nbuf
        gh[0] = pltpu.async_copy(h.at[src_i.at[0]], rows[0], gs[0])
        for i in range(NSTR):
            b = i % nbuf
            if i + 1 < NSTR:
                bn = (i + 1) % nbuf
                if sh[bn] is not None:
                    sh[bn].wait()
                gh[bn] = pltpu.async_copy(h.at[src_i.at[i + 1]], rows[bn], gs[bn])
            gh[b].wait()
            sh[b] = pltpu.async_copy(rows[b], acc.at[dst_i.at[i]], ss[b], add=True)
        for b in range(nbuf):
            if sh[b] is not None:
                sh[b].wait()
        return carry

    lax.fori_loop(0, SPT, super_body, 0)
    plsc.subcore_barrier()

    def oblk(m, carry):
        blk = m * 16 + s

        @pl.when(blk < NBLK)
        def _():
            pltpu.sync_copy(acc.at[pl.ds(blk * 128, 128)],
                            out.at[pl.ds(c * OSTR + blk * 128, 128)])

        return carry

    lax.fori_loop(0, 13, oblk, 0)


def _agg_body64(h, srcr, dstr, out, src_v, dst_v, rows_v, zbuf, acc, sem):
    W = 64
    c = lax.axis_index("c")
    s = lax.axis_index("s")
    cbase = c * HALF
    zero = jnp.zeros((16,), f32)

    def zrow(i, carry):
        for k in range(W // 16):
            zbuf[i, pl.ds(k * 16, 16)] = zero
        return carry

    lax.fori_loop(0, 128, zrow, 0)

    def zblk(m, carry):
        blk = m * 16 + s

        @pl.when(blk < NBLK_ACC)
        def _():
            pltpu.sync_copy(zbuf, acc.at[pl.ds(blk * 128, 128)])

        return carry

    lax.fori_loop(0, 13, zblk, 0)
    plsc.subcore_barrier()

    def chunk(j, carry):
        g = s * CPT64 + j
        pltpu.sync_copy(srcr.at[g], src_v)
        pltpu.sync_copy(dstr.at[g], dst_v)
        for i in range(8):
            for k in range(8):
                sl = (i, pl.ds(k * 16, 16))
                sv = src_v[sl]
                src_v[sl] = sv + jnp.where(sv >= HALF, 88, 0)
                d = dst_v[sl] - cbase
                ok = (d >= 0) & (d < HALF)
                dst_v[sl] = jnp.where(ok, d, TRASH64)
        for i in range(8):
            pltpu.async_copy(h.at[src_v.at[i]], rows_v, sem).wait()
            pltpu.sync_copy(rows_v, acc.at[dst_v.at[i]], add=True)
        return carry

    lax.fori_loop(0, CPT64, chunk, 0)
    plsc.subcore_barrier()

    def oblk(m, carry):
        blk = m * 16 + s

        @pl.when(blk < NBLK)
        def _():
            pltpu.sync_copy(acc.at[pl.ds(blk * 128, 128)],
                            out.at[pl.ds(c * OSTR + blk * 128, 128)])

        return carry

    lax.fori_loop(0, 13, oblk, 0)


def _make_agg(W):
    mesh = plsc.VectorSubcoreMesh(core_axis_name="c", subcore_axis_name="s")

    if W == 16:
        NSTR = 16
        SPT = EPT // (NSTR * 128) // 16

        @functools.partial(
            pl.kernel,
            out_type=jax.ShapeDtypeStruct((NPAD, W), f32),
            mesh=mesh,
            compiler_params=pltpu.CompilerParams(use_tc_tiling_on_sc=False),
            scratch_types=[
                pltpu.VMEM((NSTR, 128), jnp.int32),
                pltpu.VMEM((NSTR, 128), jnp.int32),
            ] + [pltpu.VMEM((128, W), f32) for _ in range(NBUF)] + [
                pltpu.VMEM_SHARED((ACC_ROWS, W), f32),
            ] + [pltpu.SemaphoreType.DMA for _ in range(2 * NBUF)],
        )
        def agg(h, srcr, dstr, out, src_i, dst_i, r0, r1, r2, acc,
                g0, g1, g2, s0, s1, s2):
            _agg_body16(NSTR, SPT, h, srcr, dstr, out, src_i, dst_i,
                        [r0, r1, r2], acc, [g0, g1, g2], [s0, s1, s2])
    else:
        @functools.partial(
            pl.kernel,
            out_type=jax.ShapeDtypeStruct((NPAD, W), f32),
            mesh=mesh,
            compiler_params=pltpu.CompilerParams(use_tc_tiling_on_sc=False),
            scratch_types=[
                pltpu.VMEM((8, 128), jnp.int32),
                pltpu.VMEM((8, 128), jnp.int32),
                pltpu.VMEM((128, W), f32),
                pltpu.VMEM((128, W), f32),
                pltpu.VMEM_SHARED((ACC_ROWS, W), f32),
                pltpu.SemaphoreType.DMA,
            ],
        )
        def agg(h, srcr, dstr, out, src_v, dst_v, rows_v, zbuf, acc, sem):
            _agg_body64(h, srcr, dstr, out, src_v, dst_v, rows_v, zbuf, acc, sem)

    return agg


_agg16 = _make_agg(16)
_agg64 = _make_agg(64)

BLK = 3584
GRID = NPAD // BLK


def _row_spec(w):
    return pl.BlockSpec((BLK, w), lambda i: (i, 0))


def _full_spec(shape):
    return pl.BlockSpec(shape, lambda i: (0,) * len(shape))


def _l0_body(acc_ref, xp_ref, wl_ref, bl_ref, wr_ref, h_ref, inv_ref):
    acc = acc_ref[...]
    inv = 1.0 / jnp.maximum(acc[:, 13:14], 1.0)
    mean = acc * inv
    o = jnp.dot(mean, wl_ref[...], preferred_element_type=f32)
    o = o + jnp.dot(xp_ref[...], wr_ref[...], preferred_element_type=f32)
    o = o + bl_ref[...][0:1, :]
    h_ref[...] = jnp.maximum(o, 0.0)
    inv_ref[...] = inv


_l0_call = pl.pallas_call(
    _l0_body,
    grid=(GRID,),
    in_specs=[_row_spec(16), _row_spec(16), _full_spec((16, 64)),
              _full_spec((8, 64)), _full_spec((16, 64))],
    out_specs=[_row_spec(64), _row_spec(1)],
    out_shape=[jax.ShapeDtypeStruct((NPAD, 64), f32),
               jax.ShapeDtypeStruct((NPAD, 1), f32)],
)


def _mid_body(acc_ref, h_ref, inv_ref, wl_ref, bl_ref, wr_ref, o_ref):
    mean = acc_ref[...] * inv_ref[...]
    o = jnp.dot(mean, wl_ref[...], preferred_element_type=f32)
    o = o + jnp.dot(h_ref[...], wr_ref[...], preferred_element_type=f32)
    o = o + bl_ref[...][0:1, :]
    o_ref[...] = jnp.maximum(o, 0.0)


_mid_call = pl.pallas_call(
    _mid_body,
    grid=(GRID,),
    in_specs=[_row_spec(64), _row_spec(64), _row_spec(1), _full_spec((64, 64)),
              _full_spec((8, 64)), _full_spec((64, 64))],
    out_specs=_row_spec(64),
    out_shape=jax.ShapeDtypeStruct((NPAD, 64), f32),
)


def _l3_body(acc_ref, h_ref, inv_ref, wl_ref, bl_ref, wr_ref, wc_ref, bc_ref,
             o_ref, pw_ref):
    mean = acc_ref[...] * inv_ref[...]
    o = jnp.dot(mean, wl_ref[...], preferred_element_type=f32)
    o = o + jnp.dot(h_ref[...], wr_ref[...], preferred_element_type=f32)
    o = o + bl_ref[...][0:1, :]
    o = jnp.maximum(o, 0.0)
    o_ref[...] = o
    pw_ref[...] = jnp.dot(o, wc_ref[...], preferred_element_type=f32) + bc_ref[...][0:1, :]


_l3_call = pl.pallas_call(
    _l3_body,
    grid=(GRID,),
    in_specs=[_row_spec(64), _row_spec(64), _row_spec(1), _full_spec((64, 64)),
              _full_spec((8, 64)), _full_spec((64, 64)), _full_spec((64, 16)),
              _full_spec((8, 16))],
    out_specs=[_row_spec(64), _row_spec(16)],
    out_shape=[jax.ShapeDtypeStruct((NPAD, 64), f32),
               jax.ShapeDtypeStruct((NPAD, 16), f32)],
)


def _fin_body(acc_ref, pw_ref, inv_ref, o_ref):
    o_ref[...] = jax.nn.sigmoid(
        acc_ref[...][:, 0:1] * inv_ref[...] + pw_ref[...][:, 1:2])


_fin_call = pl.pallas_call(
    _fin_body,
    grid=(GRID,),
    in_specs=[_row_spec(16), _row_spec(16), _row_spec(1)],
    out_specs=_row_spec(1),
    out_shape=jax.ShapeDtypeStruct((NPAD, 1), f32),
)


def _pad_rows(a):
    w = a.shape[1]
    return (jnp.zeros((2, OSTR, w), f32)
            .at[:, :HALF].set(a.reshape(2, HALF, w))
            .reshape(NPAD, w))


def kernel(x, edge_index, Wl0, bl0, Wr0, Wl1, bl1, Wr1, Wl2, bl2, Wr2,
           Wl3, bl3, Wr3, Wl4, bl4, Wr4):
    src = edge_index[0]
    dst = edge_index[1]
    pad = EPT - E
    srcf = jnp.concatenate([src, jnp.zeros((pad,), jnp.int32)])
    dstf = jnp.concatenate([dst, jnp.full((pad,), -1, jnp.int32)])
    srcr16 = srcf.reshape(-1, 16, 128)
    dstr16 = dstf.reshape(-1, 16, 128)
    pad64 = EPT64 - E
    srcr64 = jnp.concatenate([src, jnp.zeros((pad64,), jnp.int32)]).reshape(-1, 8, 128)
    dstr64 = jnp.concatenate([dst, jnp.full((pad64,), -1, jnp.int32)]).reshape(-1, 8, 128)
    xp = jnp.concatenate([x, jnp.ones((N, 1), f32), jnp.zeros((N, 2), f32)], axis=1)
    xpp = _pad_rows(xp)
    wl0p = jnp.concatenate([Wl0, jnp.zeros((3, 64), f32)], axis=0)
    wr0p = jnp.concatenate([Wr0, jnp.zeros((3, 64), f32)], axis=0)
    wcat = jnp.concatenate([Wl4, Wr4, jnp.zeros((64, 14), f32)], axis=1)
    bcat = jnp.zeros((16,), f32).at[1].set(bl4[0])
    b8 = lambda b: jnp.tile(b.reshape(1, -1), (8, 1))

    acc0 = _agg16(xpp, srcr16, dstr16)
    h0, inv = _l0_call(acc0, xpp, wl0p, b8(bl0), wr0p)
    acc1 = _agg64(h0, srcr64, dstr64)
    h1 = _mid_call(acc1, h0, inv, Wl1, b8(bl1), Wr1)
    acc2 = _agg64(h1, srcr64, dstr64)
    h2 = _mid_call(acc2, h1, inv, Wl2, b8(bl2), Wr2)
    acc3 = _agg64(h2, srcr64, dstr64)
    h3, pw = _l3_call(acc3, h2, inv, Wl3, b8(bl3), Wr3, wcat, b8(bcat))
    acc4 = _agg16(pw, srcr16, dstr16)
    outp = _fin_call(acc4, pw, inv)
    return outp.reshape(2, OSTR, 1)[:, :HALF].reshape(N, 1)
